# baseline (device time: 49063 ns/iter reference)
import jax
import jax.numpy as jnp
from jax import lax
from jax.experimental import pallas as pl
from jax.experimental.pallas import tpu as pltpu

N_DEV = 16
B = 2
SQ = 128
HQ = 4
DH = 64
DM = HQ * DH
DX = 512
SKV = N_DEV * SQ
NG = 32
SCALE = 0.125
NEG = -1e9
PW = 384


def _merge_packed(parts):
    m_all = [p[:, DM:DM + HQ] for p in parts]
    l_all = [p[:, DM + HQ:DM + 2 * HQ] for p in parts]
    M = m_all[0]
    for mm in m_all[1:]:
        M = jnp.maximum(M, mm)
    coefs = [jnp.exp(mm - M) for mm in m_all]
    L = coefs[0] * l_all[0]
    for cf, ll in zip(coefs[1:], l_all[1:]):
        L = L + cf * ll
    acc_heads = []
    for hh in range(HQ):
        num = parts[0][:, hh * DH:(hh + 1) * DH] * coefs[0][:, hh:hh + 1]
        for p_, cf in zip(parts[1:], coefs[1:]):
            num = num + p_[:, hh * DH:(hh + 1) * DH] * cf[:, hh:hh + 1]
        acc_heads.append(num)
    return acc_heads, M, L


def _pack(acc_heads, M, L):
    return jnp.concatenate(
        acc_heads + [M, L, jnp.zeros((NG, PW - DM - 2 * HQ), jnp.float32)],
        axis=1)


def kernel(x, Wq, K_ext, V_ext, Wo):
    k2 = K_ext.reshape(B, SQ, DM)
    v2 = V_ext.reshape(B, SQ, DM)

    def body(x_ref, wq_ref, k_ref, v_ref, wo_ref, out_ref,
             kvown, halobuf, gbuf, pown, pplane, pprt,
             halo_send, halo_recv, gsend, grecv,
             plsend, plrecv, psend, precv):
        my = lax.axis_index("i")
        in4 = lax.rem(my, 4)
        lead = my - in4
        MESH = pl.DeviceIdType.MESH

        bsem = pltpu.get_barrier_semaphore()

        @pl.when(my == 0)
        def _():
            for q in range(1, N_DEV):
                pl.semaphore_signal(bsem, inc=1, device_id=(q,),
                                    device_id_type=MESH)

        @pl.when(my != 0)
        def _():
            pl.semaphore_signal(bsem, inc=1, device_id=(0,),
                                device_id_type=MESH)

        @pl.when(my > 1)
        def _():
            pl.semaphore_signal(bsem, inc=1, device_id=(my - 1,),
                                device_id_type=MESH)

        @pl.when((my > 0) & (my < N_DEV - 1))
        def _():
            pl.semaphore_signal(bsem, inc=1, device_id=(my + 1,),
                                device_id_type=MESH)

        @pl.when((in4 == 0) & (my > 0))
        def _():
            pl.semaphore_signal(bsem, inc=1, device_id=(my + 2,),
                                device_id_type=MESH)
            pl.semaphore_signal(bsem, inc=1, device_id=(my + 3,),
                                device_id_type=MESH)

        @pl.when(my == 0)
        def _():
            pl.semaphore_wait(bsem, N_DEV - 1)

        @pl.when(my == 1)
        def _():
            pl.semaphore_wait(bsem, 2)

        four = (my >= 6) & (in4 >= 2) & (my != N_DEV - 1)

        @pl.when(four)
        def _():
            pl.semaphore_wait(bsem, 4)

        @pl.when((my > 1) & jnp.logical_not(four))
        def _():
            pl.semaphore_wait(bsem, 3)

        kvown[:, :, 0:DM] = k_ref[...]
        kvown[:, :, DM:] = v_ref[...]

        @pl.when(my == 0)
        def _():
            halobuf[0] = jnp.zeros((B, SQ, 2 * DM), jnp.float32)

        @pl.when(my == N_DEV - 1)
        def _():
            halobuf[1] = jnp.zeros((B, SQ, 2 * DM), jnp.float32)

        @pl.when(my < N_DEV - 1)
        def _():
            pltpu.make_async_remote_copy(
                src_ref=kvown, dst_ref=halobuf.at[0],
                send_sem=halo_send.at[0], recv_sem=halo_recv.at[0],
                device_id=(my + 1,), device_id_type=MESH).start()

        @pl.when(my > 0)
        def _():
            pltpu.make_async_remote_copy(
                src_ref=kvown, dst_ref=halobuf.at[1],
                send_sem=halo_send.at[1], recv_sem=halo_recv.at[1],
                device_id=(my - 1,), device_id_type=MESH).start()

        q_all = [jnp.dot(x_ref[b], wq_ref[...],
                         preferred_element_type=jnp.float32)
                 for b in range(B)]

        @pl.when(my == 0)
        def _():
            for b in range(B):
                gbuf[0, b] = q_all[b][0:NG, :]
                gbuf[1, b] = k_ref[b, 0:NG, :]
                gbuf[2, b] = v_ref[b, 0:NG, :]
            for q in range(1, N_DEV):
                pltpu.make_async_remote_copy(
                    src_ref=gbuf, dst_ref=gbuf,
                    send_sem=gsend.at[q - 1], recv_sem=grecv.at[0],
                    device_id=(q,), device_id_type=MESH).start()

        @pl.when(my != 0)
        def _():
            pltpu.make_async_remote_copy(
                src_ref=gbuf, dst_ref=gbuf,
                send_sem=gsend.at[0], recv_sem=grecv.at[0],
                device_id=(0,), device_id_type=MESH).wait_recv()

        for b in range(B):
            accs, ms, ls = [], [], []
            for hh in range(HQ):
                qg_h = gbuf[0, b][:, hh * DH:(hh + 1) * DH]
                k_h = k_ref[b][:, hh * DH:(hh + 1) * DH]
                v_h = v_ref[b][:, hh * DH:(hh + 1) * DH]
                s = lax.dot_general(
                    qg_h, k_h, (((1,), (1,)), ((), ())),
                    preferred_element_type=jnp.float32) * SCALE
                m = jnp.max(s, axis=1, keepdims=True)
                w = jnp.exp(s - m)
                ls.append(jnp.sum(w, axis=1, keepdims=True))
                ms.append(m)
                accs.append(jnp.dot(w, v_h,
                                    preferred_element_type=jnp.float32))
            pown[b] = jnp.concatenate(
                accs + ms + ls + [jnp.zeros((NG, PW - DM - 2 * HQ),
                                            jnp.float32)], axis=1)

        @pl.when(in4 != 0)
        def _():
            pltpu.make_async_remote_copy(
                src_ref=pown, dst_ref=pplane.at[in4 - 1],
                send_sem=plsend.at[0], recv_sem=plrecv.at[in4 - 1],
                device_id=(lead,), device_id_type=MESH).start()

        @pl.when(my > 0)
        def _():
            pltpu.make_async_remote_copy(
                src_ref=kvown, dst_ref=halobuf.at[0],
                send_sem=halo_send.at[0], recv_sem=halo_recv.at[0],
                device_id=(my,), device_id_type=MESH).wait_recv()

        @pl.when(my < N_DEV - 1)
        def _():
            pltpu.make_async_remote_copy(
                src_ref=kvown, dst_ref=halobuf.at[1],
                send_sem=halo_send.at[1], recv_sem=halo_recv.at[1],
                device_id=(my,), device_id_type=MESH).wait_recv()

        r = lax.broadcasted_iota(jnp.int32, (SQ, 4 * SQ), 0)
        c = lax.broadcasted_iota(jnp.int32, (SQ, 4 * SQ), 1)
        qi = my * SQ + r
        ki = (my - 1) * SQ + c
        band_ok = ((c < 3 * SQ) & (ki >= 0) & (ki < SKV)
                   & ((jnp.abs(qi - ki) <= 128) | (ki < NG) | (qi < NG)))
        glob_ok = (c >= 3 * SQ) & (c < 3 * SQ + NG) & (my >= 2)
        mask = band_ok | glob_ok

        zpad = jnp.zeros((SQ - NG, DM), jnp.float32)
        for b in range(B):
            k_cat = jnp.concatenate(
                [halobuf[0, b][:, :DM], k_ref[b], halobuf[1, b][:, :DM],
                 gbuf[1, b], zpad], axis=0)
            v_cat = jnp.concatenate(
                [halobuf[0, b][:, DM:], v_ref[b], halobuf[1, b][:, DM:],
                 gbuf[2, b], zpad], axis=0)
            ctx_heads = []
            for hh in range(HQ):
                q_h = q_all[b][:, hh * DH:(hh + 1) * DH]
                k_h = k_cat[:, hh * DH:(hh + 1) * DH]
                v_h = v_cat[:, hh * DH:(hh + 1) * DH]
                s = lax.dot_general(
                    q_h, k_h, (((1,), (1,)), ((), ())),
                    preferred_element_type=jnp.float32) * SCALE
                s = jnp.where(mask, s, NEG)
                m = jnp.max(s, axis=1, keepdims=True)
                w = jnp.exp(s - m)
                w = w / jnp.sum(w, axis=1, keepdims=True)
                ctx_heads.append(jnp.dot(w, v_h,
                                         preferred_element_type=jnp.float32))
            ctx = jnp.concatenate(ctx_heads, axis=-1)
            out_ref[b] = jnp.dot(ctx, wo_ref[...],
                                 preferred_element_type=jnp.float32)

        @pl.when(in4 == 0)
        def _():
            for j in range(3):
                pltpu.make_async_remote_copy(
                    src_ref=pown, dst_ref=pplane.at[j],
                    send_sem=plsend.at[0], recv_sem=plrecv.at[j],
                    device_id=(my,), device_id_type=MESH).wait_recv()

        @pl.when((in4 == 0) & (my > 0))
        def _():
            for b in range(B):
                acc_h, M, L = _merge_packed(
                    [pown[b]] + [pplane[j, b] for j in range(3)])
                pown[b] = _pack(acc_h, M, L)
            pltpu.make_async_remote_copy(
                src_ref=pown, dst_ref=pprt.at[my // 4 - 1],
                send_sem=psend.at[0], recv_sem=precv.at[my // 4 - 1],
                device_id=(0,), device_id_type=MESH).start()

        @pl.when(my == 0)
        def _():
            for j in range(3):
                pltpu.make_async_remote_copy(
                    src_ref=pown, dst_ref=pprt.at[j],
                    send_sem=psend.at[0], recv_sem=precv.at[j],
                    device_id=(my,), device_id_type=MESH).wait_recv()
            for b in range(B):
                plane0_acc, M0, L0 = _merge_packed(
                    [pown[b]] + [pplane[j, b] for j in range(3)])
                acc_h, M, L = _merge_packed(
                    [_pack(plane0_acc, M0, L0)]
                    + [pprt[j, b] for j in range(3)])
                ctx_g = jnp.concatenate(
                    [acc_h[hh] / L[:, hh:hh + 1] for hh in range(HQ)],
                    axis=-1)
                out_ref[b, 0:NG] = jnp.dot(ctx_g, wo_ref[...],
                                           preferred_element_type=jnp.float32)

        @pl.when(my < N_DEV - 1)
        def _():
            pltpu.make_async_remote_copy(
                src_ref=kvown, dst_ref=halobuf.at[0],
                send_sem=halo_send.at[0], recv_sem=halo_recv.at[0],
                device_id=(my,), device_id_type=MESH).wait_send()

        @pl.when(my > 0)
        def _():
            pltpu.make_async_remote_copy(
                src_ref=kvown, dst_ref=halobuf.at[1],
                send_sem=halo_send.at[1], recv_sem=halo_recv.at[1],
                device_id=(my,), device_id_type=MESH).wait_send()

        @pl.when(in4 != 0)
        def _():
            pltpu.make_async_remote_copy(
                src_ref=pown, dst_ref=pplane.at[0],
                send_sem=plsend.at[0], recv_sem=plrecv.at[0],
                device_id=(my,), device_id_type=MESH).wait_send()

        @pl.when((in4 == 0) & (my > 0))
        def _():
            pltpu.make_async_remote_copy(
                src_ref=pown, dst_ref=pprt.at[0],
                send_sem=psend.at[0], recv_sem=precv.at[0],
                device_id=(my,), device_id_type=MESH).wait_send()

        @pl.when(my == 0)
        def _():
            for q in range(1, N_DEV):
                pltpu.make_async_remote_copy(
                    src_ref=gbuf, dst_ref=gbuf,
                    send_sem=gsend.at[q - 1], recv_sem=grecv.at[0],
                    device_id=(my,), device_id_type=MESH).wait_send()

    return pl.pallas_call(
        body,
        out_shape=jax.ShapeDtypeStruct((B, SQ, DX), jnp.float32),
        in_specs=[pl.BlockSpec(memory_space=pltpu.VMEM)] * 5,
        out_specs=pl.BlockSpec(memory_space=pltpu.VMEM),
        scratch_shapes=[
            pltpu.VMEM((B, SQ, 2 * DM), jnp.float32),
            pltpu.VMEM((2, B, SQ, 2 * DM), jnp.float32),
            pltpu.VMEM((3, B, NG, DM), jnp.float32),
            pltpu.VMEM((B, NG, PW), jnp.float32),
            pltpu.VMEM((3, B, NG, PW), jnp.float32),
            pltpu.VMEM((3, B, NG, PW), jnp.float32),
            pltpu.SemaphoreType.DMA((2,)),
            pltpu.SemaphoreType.DMA((2,)),
            pltpu.SemaphoreType.DMA((N_DEV - 1,)),
            pltpu.SemaphoreType.DMA((1,)),
            pltpu.SemaphoreType.DMA((1,)),
            pltpu.SemaphoreType.DMA((3,)),
            pltpu.SemaphoreType.DMA((1,)),
            pltpu.SemaphoreType.DMA((3,)),
        ],
        compiler_params=pltpu.CompilerParams(collective_id=0),
    )(x, Wq, k2, v2, Wo)


# device time: 43725 ns/iter; 1.1221x vs baseline; 1.1221x over previous
import jax
import jax.numpy as jnp
from jax import lax
from jax.experimental import pallas as pl
from jax.experimental.pallas import tpu as pltpu

N_DEV = 16
B = 2
SQ = 128
HQ = 4
DH = 64
DM = HQ * DH
DX = 512
SKV = N_DEV * SQ
NG = 32
SCALE = 0.125
NEG = -1e9
PW = 384


def _merge_packed(parts):
    m_all = [p[:, DM:DM + HQ] for p in parts]
    l_all = [p[:, DM + HQ:DM + 2 * HQ] for p in parts]
    M = m_all[0]
    for mm in m_all[1:]:
        M = jnp.maximum(M, mm)
    coefs = [jnp.exp(mm - M) for mm in m_all]
    L = coefs[0] * l_all[0]
    for cf, ll in zip(coefs[1:], l_all[1:]):
        L = L + cf * ll
    acc_heads = []
    for hh in range(HQ):
        num = parts[0][:, hh * DH:(hh + 1) * DH] * coefs[0][:, hh:hh + 1]
        for p_, cf in zip(parts[1:], coefs[1:]):
            num = num + p_[:, hh * DH:(hh + 1) * DH] * cf[:, hh:hh + 1]
        acc_heads.append(num)
    return acc_heads, M, L


def _pack(acc_heads, M, L):
    return jnp.concatenate(
        acc_heads + [M, L, jnp.zeros((NG, PW - DM - 2 * HQ), jnp.float32)],
        axis=1)


def kernel(x, Wq, K_ext, V_ext, Wo):
    k2 = K_ext.reshape(B, SQ, DM)
    v2 = V_ext.reshape(B, SQ, DM)

    def body(x_ref, wq_ref, k_ref, v_ref, wo_ref, out_ref,
             kvown, halobuf, gq, gkv, pown, pplane, pprt,
             halo_send, halo_recv, gqsend, gqrecv, gkvsend, gkvrecv,
             plsend, plrecv, psend, precv):
        my = lax.axis_index("i")
        in4 = lax.rem(my, 4)
        lead = my - in4
        MESH = pl.DeviceIdType.MESH

        bsem = pltpu.get_barrier_semaphore()

        @pl.when(my == 0)
        def _():
            for q in range(1, N_DEV):
                pl.semaphore_signal(bsem, inc=1, device_id=(q,),
                                    device_id_type=MESH)

        @pl.when(my != 0)
        def _():
            pl.semaphore_signal(bsem, inc=1, device_id=(0,),
                                device_id_type=MESH)

        @pl.when(my > 1)
        def _():
            pl.semaphore_signal(bsem, inc=1, device_id=(my - 1,),
                                device_id_type=MESH)

        @pl.when((my > 0) & (my < N_DEV - 1))
        def _():
            pl.semaphore_signal(bsem, inc=1, device_id=(my + 1,),
                                device_id_type=MESH)

        @pl.when((in4 == 0) & (my > 0))
        def _():
            pl.semaphore_signal(bsem, inc=1, device_id=(my + 2,),
                                device_id_type=MESH)
            pl.semaphore_signal(bsem, inc=1, device_id=(my + 3,),
                                device_id_type=MESH)

        @pl.when(my == 0)
        def _():
            pl.semaphore_wait(bsem, N_DEV - 1)

        @pl.when(my == 1)
        def _():
            pl.semaphore_wait(bsem, 2)

        four = (my >= 6) & (in4 >= 2) & (my != N_DEV - 1)

        @pl.when(four)
        def _():
            pl.semaphore_wait(bsem, 4)

        @pl.when((my > 1) & jnp.logical_not(four))
        def _():
            pl.semaphore_wait(bsem, 3)

        @pl.when(my == 0)
        def _():
            qg_rows = [jnp.dot(x_ref[b][0:NG, :], wq_ref[...],
                               preferred_element_type=jnp.float32)
                       for b in range(B)]
            for b in range(B):
                gq[b] = qg_rows[b]
            for q in range(1, N_DEV):
                pltpu.make_async_remote_copy(
                    src_ref=gq, dst_ref=gq,
                    send_sem=gqsend.at[q - 1], recv_sem=gqrecv.at[0],
                    device_id=(q,), device_id_type=MESH).start()

        kvown[:, :, 0:DM] = k_ref[...]
        kvown[:, :, DM:] = v_ref[...]

        @pl.when(my == 0)
        def _():
            halobuf[0] = jnp.zeros((B, SQ, 2 * DM), jnp.float32)

        @pl.when(my == N_DEV - 1)
        def _():
            halobuf[1] = jnp.zeros((B, SQ, 2 * DM), jnp.float32)

        @pl.when(my < N_DEV - 1)
        def _():
            pltpu.make_async_remote_copy(
                src_ref=kvown, dst_ref=halobuf.at[0],
                send_sem=halo_send.at[0], recv_sem=halo_recv.at[0],
                device_id=(my + 1,), device_id_type=MESH).start()

        @pl.when(my > 0)
        def _():
            pltpu.make_async_remote_copy(
                src_ref=kvown, dst_ref=halobuf.at[1],
                send_sem=halo_send.at[1], recv_sem=halo_recv.at[1],
                device_id=(my - 1,), device_id_type=MESH).start()

        q_all = [jnp.dot(x_ref[b], wq_ref[...],
                         preferred_element_type=jnp.float32)
                 for b in range(B)]

        @pl.when(my == 0)
        def _():
            for b in range(B):
                gkv[0, b] = k_ref[b, 0:NG, :]
                gkv[1, b] = v_ref[b, 0:NG, :]
            for q in range(1, N_DEV):
                pltpu.make_async_remote_copy(
                    src_ref=gkv, dst_ref=gkv,
                    send_sem=gkvsend.at[q - 1], recv_sem=gkvrecv.at[0],
                    device_id=(q,), device_id_type=MESH).start()

        @pl.when(my != 0)
        def _():
            pltpu.make_async_remote_copy(
                src_ref=gq, dst_ref=gq,
                send_sem=gqsend.at[0], recv_sem=gqrecv.at[0],
                device_id=(0,), device_id_type=MESH).wait_recv()

        for b in range(B):
            accs, ms, ls = [], [], []
            for hh in range(HQ):
                qg_h = gq[b][:, hh * DH:(hh + 1) * DH]
                k_h = k_ref[b][:, hh * DH:(hh + 1) * DH]
                v_h = v_ref[b][:, hh * DH:(hh + 1) * DH]
                s = lax.dot_general(
                    qg_h, k_h, (((1,), (1,)), ((), ())),
                    preferred_element_type=jnp.float32) * SCALE
                m = jnp.max(s, axis=1, keepdims=True)
                w = jnp.exp(s - m)
                ls.append(jnp.sum(w, axis=1, keepdims=True))
                ms.append(m)
                accs.append(jnp.dot(w, v_h,
                                    preferred_element_type=jnp.float32))
            pown[b] = jnp.concatenate(
                accs + ms + ls + [jnp.zeros((NG, PW - DM - 2 * HQ),
                                            jnp.float32)], axis=1)

        @pl.when(in4 != 0)
        def _():
            pltpu.make_async_remote_copy(
                src_ref=pown, dst_ref=pplane.at[in4 - 1],
                send_sem=plsend.at[0], recv_sem=plrecv.at[in4 - 1],
                device_id=(lead,), device_id_type=MESH).start()

        @pl.when(my > 0)
        def _():
            pltpu.make_async_remote_copy(
                src_ref=kvown, dst_ref=halobuf.at[0],
                send_sem=halo_send.at[0], recv_sem=halo_recv.at[0],
                device_id=(my,), device_id_type=MESH).wait_recv()

        @pl.when(my < N_DEV - 1)
        def _():
            pltpu.make_async_remote_copy(
                src_ref=kvown, dst_ref=halobuf.at[1],
                send_sem=halo_send.at[1], recv_sem=halo_recv.at[1],
                device_id=(my,), device_id_type=MESH).wait_recv()

        @pl.when(my != 0)
        def _():
            pltpu.make_async_remote_copy(
                src_ref=gkv, dst_ref=gkv,
                send_sem=gkvsend.at[0], recv_sem=gkvrecv.at[0],
                device_id=(0,), device_id_type=MESH).wait_recv()

        r = lax.broadcasted_iota(jnp.int32, (SQ, 4 * SQ), 0)
        c = lax.broadcasted_iota(jnp.int32, (SQ, 4 * SQ), 1)
        qi = my * SQ + r
        ki = (my - 1) * SQ + c
        band_ok = ((c < 3 * SQ) & (ki >= 0) & (ki < SKV)
                   & ((jnp.abs(qi - ki) <= 128) | (ki < NG) | (qi < NG)))
        glob_ok = (c >= 3 * SQ) & (c < 3 * SQ + NG) & (my >= 2)
        mask = band_ok | glob_ok

        zpad = jnp.zeros((SQ - NG, DM), jnp.float32)
        for b in range(B):
            k_cat = jnp.concatenate(
                [halobuf[0, b][:, :DM], k_ref[b], halobuf[1, b][:, :DM],
                 gkv[0, b], zpad], axis=0)
            v_cat = jnp.concatenate(
                [halobuf[0, b][:, DM:], v_ref[b], halobuf[1, b][:, DM:],
                 gkv[1, b], zpad], axis=0)
            ctx_heads = []
            for hh in range(HQ):
                q_h = q_all[b][:, hh * DH:(hh + 1) * DH]
                k_h = k_cat[:, hh * DH:(hh + 1) * DH]
                v_h = v_cat[:, hh * DH:(hh + 1) * DH]
                s = lax.dot_general(
                    q_h, k_h, (((1,), (1,)), ((), ())),
                    preferred_element_type=jnp.float32) * SCALE
                s = jnp.where(mask, s, NEG)
                m = jnp.max(s, axis=1, keepdims=True)
                w = jnp.exp(s - m)
                w = w / jnp.sum(w, axis=1, keepdims=True)
                ctx_heads.append(jnp.dot(w, v_h,
                                         preferred_element_type=jnp.float32))
            ctx = jnp.concatenate(ctx_heads, axis=-1)
            out_ref[b] = jnp.dot(ctx, wo_ref[...],
                                 preferred_element_type=jnp.float32)

        @pl.when(in4 == 0)
        def _():
            for j in range(3):
                pltpu.make_async_remote_copy(
                    src_ref=pown, dst_ref=pplane.at[j],
                    send_sem=plsend.at[0], recv_sem=plrecv.at[j],
                    device_id=(my,), device_id_type=MESH).wait_recv()

        @pl.when((in4 == 0) & (my > 0))
        def _():
            for b in range(B):
                acc_h, M, L = _merge_packed(
                    [pown[b]] + [pplane[j, b] for j in range(3)])
                pown[b] = _pack(acc_h, M, L)
            pltpu.make_async_remote_copy(
                src_ref=pown, dst_ref=pprt.at[my // 4 - 1],
                send_sem=psend.at[0], recv_sem=precv.at[my // 4 - 1],
                device_id=(0,), device_id_type=MESH).start()

        @pl.when(my == 0)
        def _():
            for j in range(3):
                pltpu.make_async_remote_copy(
                    src_ref=pown, dst_ref=pprt.at[j],
                    send_sem=psend.at[0], recv_sem=precv.at[j],
                    device_id=(my,), device_id_type=MESH).wait_recv()
            for b in range(B):
                plane0_acc, M0, L0 = _merge_packed(
                    [pown[b]] + [pplane[j, b] for j in range(3)])
                acc_h, M, L = _merge_packed(
                    [_pack(plane0_acc, M0, L0)]
                    + [pprt[j, b] for j in range(3)])
                ctx_g = jnp.concatenate(
                    [acc_h[hh] / L[:, hh:hh + 1] for hh in range(HQ)],
                    axis=-1)
                out_ref[b, 0:NG] = jnp.dot(ctx_g, wo_ref[...],
                                           preferred_element_type=jnp.float32)

        @pl.when(my < N_DEV - 1)
        def _():
            pltpu.make_async_remote_copy(
                src_ref=kvown, dst_ref=halobuf.at[0],
                send_sem=halo_send.at[0], recv_sem=halo_recv.at[0],
                device_id=(my,), device_id_type=MESH).wait_send()

        @pl.when(my > 0)
        def _():
            pltpu.make_async_remote_copy(
                src_ref=kvown, dst_ref=halobuf.at[1],
                send_sem=halo_send.at[1], recv_sem=halo_recv.at[1],
                device_id=(my,), device_id_type=MESH).wait_send()

        @pl.when(in4 != 0)
        def _():
            pltpu.make_async_remote_copy(
                src_ref=pown, dst_ref=pplane.at[0],
                send_sem=plsend.at[0], recv_sem=plrecv.at[0],
                device_id=(my,), device_id_type=MESH).wait_send()

        @pl.when((in4 == 0) & (my > 0))
        def _():
            pltpu.make_async_remote_copy(
                src_ref=pown, dst_ref=pprt.at[0],
                send_sem=psend.at[0], recv_sem=precv.at[0],
                device_id=(my,), device_id_type=MESH).wait_send()

        @pl.when(my == 0)
        def _():
            for q in range(1, N_DEV):
                pltpu.make_async_remote_copy(
                    src_ref=gq, dst_ref=gq,
                    send_sem=gqsend.at[q - 1], recv_sem=gqrecv.at[0],
                    device_id=(my,), device_id_type=MESH).wait_send()
                pltpu.make_async_remote_copy(
                    src_ref=gkv, dst_ref=gkv,
                    send_sem=gkvsend.at[q - 1], recv_sem=gkvrecv.at[0],
                    device_id=(my,), device_id_type=MESH).wait_send()

    return pl.pallas_call(
        body,
        out_shape=jax.ShapeDtypeStruct((B, SQ, DX), jnp.float32),
        in_specs=[pl.BlockSpec(memory_space=pltpu.VMEM)] * 5,
        out_specs=pl.BlockSpec(memory_space=pltpu.VMEM),
        scratch_shapes=[
            pltpu.VMEM((B, SQ, 2 * DM), jnp.float32),
            pltpu.VMEM((2, B, SQ, 2 * DM), jnp.float32),
            pltpu.VMEM((B, NG, DM), jnp.float32),
            pltpu.VMEM((2, B, NG, DM), jnp.float32),
            pltpu.VMEM((B, NG, PW), jnp.float32),
            pltpu.VMEM((3, B, NG, PW), jnp.float32),
            pltpu.VMEM((3, B, NG, PW), jnp.float32),
            pltpu.SemaphoreType.DMA((2,)),
            pltpu.SemaphoreType.DMA((2,)),
            pltpu.SemaphoreType.DMA((N_DEV - 1,)),
            pltpu.SemaphoreType.DMA((1,)),
            pltpu.SemaphoreType.DMA((N_DEV - 1,)),
            pltpu.SemaphoreType.DMA((1,)),
            pltpu.SemaphoreType.DMA((1,)),
            pltpu.SemaphoreType.DMA((3,)),
            pltpu.SemaphoreType.DMA((1,)),
            pltpu.SemaphoreType.DMA((3,)),
        ],
        compiler_params=pltpu.CompilerParams(collective_id=0),
    )(x, Wq, k2, v2, Wo)


# device time: 29162 ns/iter; 1.6824x vs baseline; 1.4994x over previous
import jax
import jax.numpy as jnp
from jax import lax
from jax.experimental import pallas as pl
from jax.experimental.pallas import tpu as pltpu

N_DEV = 16
B = 2
SQ = 128
HQ = 4
DH = 64
DM = HQ * DH
DX = 512
SKV = N_DEV * SQ
NG = 32
SCALE = 0.125
NEG = -1e9
PW = 384


def _merge_packed(parts):
    m_all = [p[:, DM:DM + HQ] for p in parts]
    l_all = [p[:, DM + HQ:DM + 2 * HQ] for p in parts]
    M = m_all[0]
    for mm in m_all[1:]:
        M = jnp.maximum(M, mm)
    coefs = [jnp.exp(mm - M) for mm in m_all]
    L = coefs[0] * l_all[0]
    for cf, ll in zip(coefs[1:], l_all[1:]):
        L = L + cf * ll
    acc_heads = []
    for hh in range(HQ):
        num = parts[0][:, hh * DH:(hh + 1) * DH] * coefs[0][:, hh:hh + 1]
        for p_, cf in zip(parts[1:], coefs[1:]):
            num = num + p_[:, hh * DH:(hh + 1) * DH] * cf[:, hh:hh + 1]
        acc_heads.append(num)
    return acc_heads, M, L


def _pack(acc_heads, M, L):
    return jnp.concatenate(
        acc_heads + [M, L, jnp.zeros((NG, PW - DM - 2 * HQ), jnp.float32)],
        axis=1)


def kernel(x, Wq, K_ext, V_ext, Wo):
    k2 = K_ext.reshape(B, SQ, DM)
    v2 = V_ext.reshape(B, SQ, DM)

    def body(x_ref, wq_ref, k_ref, v_ref, wo_ref, out_ref,
             kvown, halobuf, gq, gkv, pown, pplane, pprt,
             halo_send, halo_recv, gqsend, gqrecv, gkvsend, gkvrecv,
             plsend, plrecv, psend, precv):
        my = lax.axis_index("i")
        in4 = lax.rem(my, 4)
        lead = my - in4
        MESH = pl.DeviceIdType.MESH

        bsem = pltpu.get_barrier_semaphore()

        @pl.when(my == 0)
        def _():
            for q in range(1, N_DEV):
                pl.semaphore_signal(bsem, inc=1, device_id=(q,),
                                    device_id_type=MESH)

        @pl.when(my != 0)
        def _():
            pl.semaphore_signal(bsem, inc=1, device_id=(0,),
                                device_id_type=MESH)

        @pl.when(my > 1)
        def _():
            pl.semaphore_signal(bsem, inc=1, device_id=(my - 1,),
                                device_id_type=MESH)

        @pl.when((my > 0) & (my < N_DEV - 1))
        def _():
            pl.semaphore_signal(bsem, inc=1, device_id=(my + 1,),
                                device_id_type=MESH)

        @pl.when((in4 == 0) & (my > 0))
        def _():
            pl.semaphore_signal(bsem, inc=1, device_id=(my + 2,),
                                device_id_type=MESH)
            pl.semaphore_signal(bsem, inc=1, device_id=(my + 3,),
                                device_id_type=MESH)

        @pl.when(my == 0)
        def _():
            pl.semaphore_wait(bsem, N_DEV - 1)

        @pl.when(my == 1)
        def _():
            pl.semaphore_wait(bsem, 2)

        four = (my >= 6) & (in4 >= 2) & (my != N_DEV - 1)

        @pl.when(four)
        def _():
            pl.semaphore_wait(bsem, 4)

        @pl.when((my > 1) & jnp.logical_not(four))
        def _():
            pl.semaphore_wait(bsem, 3)

        @pl.when(my == 0)
        def _():
            qg_rows = [jnp.dot(x_ref[b][0:NG, :], wq_ref[...],
                               preferred_element_type=jnp.float32)
                       for b in range(B)]
            for b in range(B):
                gq[b] = qg_rows[b].astype(jnp.bfloat16)
            for q in range(1, N_DEV):
                pltpu.make_async_remote_copy(
                    src_ref=gq, dst_ref=gq,
                    send_sem=gqsend.at[q - 1], recv_sem=gqrecv.at[0],
                    device_id=(q,), device_id_type=MESH).start()

        kvown[:, :, 0:DM] = k_ref[...].astype(jnp.bfloat16)
        kvown[:, :, DM:] = v_ref[...].astype(jnp.bfloat16)

        @pl.when(my == 0)
        def _():
            halobuf[0] = jnp.zeros((B, SQ, 2 * DM), jnp.bfloat16)

        @pl.when(my == N_DEV - 1)
        def _():
            halobuf[1] = jnp.zeros((B, SQ, 2 * DM), jnp.bfloat16)

        @pl.when(my < N_DEV - 1)
        def _():
            pltpu.make_async_remote_copy(
                src_ref=kvown, dst_ref=halobuf.at[0],
                send_sem=halo_send.at[0], recv_sem=halo_recv.at[0],
                device_id=(my + 1,), device_id_type=MESH).start()

        @pl.when(my > 0)
        def _():
            pltpu.make_async_remote_copy(
                src_ref=kvown, dst_ref=halobuf.at[1],
                send_sem=halo_send.at[1], recv_sem=halo_recv.at[1],
                device_id=(my - 1,), device_id_type=MESH).start()

        q_all = [jnp.dot(x_ref[b], wq_ref[...],
                         preferred_element_type=jnp.float32)
                 for b in range(B)]

        @pl.when(my == 0)
        def _():
            for b in range(B):
                gkv[0, b] = k_ref[b, 0:NG, :].astype(jnp.bfloat16)
                gkv[1, b] = v_ref[b, 0:NG, :].astype(jnp.bfloat16)
            for q in range(1, N_DEV):
                pltpu.make_async_remote_copy(
                    src_ref=gkv, dst_ref=gkv,
                    send_sem=gkvsend.at[q - 1], recv_sem=gkvrecv.at[0],
                    device_id=(q,), device_id_type=MESH).start()

        @pl.when(my != 0)
        def _():
            pltpu.make_async_remote_copy(
                src_ref=gq, dst_ref=gq,
                send_sem=gqsend.at[0], recv_sem=gqrecv.at[0],
                device_id=(0,), device_id_type=MESH).wait_recv()

        for b in range(B):
            accs, ms, ls = [], [], []
            for hh in range(HQ):
                qg_h = gq[b][:, hh * DH:(hh + 1) * DH]
                k_h = k_ref[b][:, hh * DH:(hh + 1) * DH].astype(jnp.bfloat16)
                v_h = v_ref[b][:, hh * DH:(hh + 1) * DH].astype(jnp.bfloat16)
                s = lax.dot_general(
                    qg_h, k_h, (((1,), (1,)), ((), ())),
                    preferred_element_type=jnp.float32) * SCALE
                m = jnp.max(s, axis=1, keepdims=True)
                w = jnp.exp(s - m)
                ls.append(jnp.sum(w, axis=1, keepdims=True))
                ms.append(m)
                accs.append(jnp.dot(w.astype(jnp.bfloat16), v_h,
                                    preferred_element_type=jnp.float32))
            pown[b] = jnp.concatenate(
                accs + ms + ls + [jnp.zeros((NG, PW - DM - 2 * HQ),
                                            jnp.float32)], axis=1)

        @pl.when(in4 != 0)
        def _():
            pltpu.make_async_remote_copy(
                src_ref=pown, dst_ref=pplane.at[in4 - 1],
                send_sem=plsend.at[0], recv_sem=plrecv.at[in4 - 1],
                device_id=(lead,), device_id_type=MESH).start()

        @pl.when(in4 == 0)
        def _():
            for j in range(3):
                pltpu.make_async_remote_copy(
                    src_ref=pown, dst_ref=pplane.at[j],
                    send_sem=plsend.at[0], recv_sem=plrecv.at[j],
                    device_id=(my,), device_id_type=MESH).wait_recv()

        @pl.when((in4 == 0) & (my > 0))
        def _():
            for b in range(B):
                acc_h, M, L = _merge_packed(
                    [pown[b]] + [pplane[j, b] for j in range(3)])
                pown[b] = _pack(acc_h, M, L)
            pltpu.make_async_remote_copy(
                src_ref=pown, dst_ref=pprt.at[my // 4 - 1],
                send_sem=psend.at[0], recv_sem=precv.at[my // 4 - 1],
                device_id=(0,), device_id_type=MESH).start()

        @pl.when(my > 0)
        def _():
            pltpu.make_async_remote_copy(
                src_ref=kvown, dst_ref=halobuf.at[0],
                send_sem=halo_send.at[0], recv_sem=halo_recv.at[0],
                device_id=(my,), device_id_type=MESH).wait_recv()

        @pl.when(my < N_DEV - 1)
        def _():
            pltpu.make_async_remote_copy(
                src_ref=kvown, dst_ref=halobuf.at[1],
                send_sem=halo_send.at[1], recv_sem=halo_recv.at[1],
                device_id=(my,), device_id_type=MESH).wait_recv()

        @pl.when(my != 0)
        def _():
            pltpu.make_async_remote_copy(
                src_ref=gkv, dst_ref=gkv,
                send_sem=gkvsend.at[0], recv_sem=gkvrecv.at[0],
                device_id=(0,), device_id_type=MESH).wait_recv()

        r = lax.broadcasted_iota(jnp.int32, (SQ, 4 * SQ), 0)
        c = lax.broadcasted_iota(jnp.int32, (SQ, 4 * SQ), 1)
        qi = my * SQ + r
        ki = (my - 1) * SQ + c
        band_ok = ((c < 3 * SQ) & (ki >= 0) & (ki < SKV)
                   & ((jnp.abs(qi - ki) <= 128) | (ki < NG) | (qi < NG)))
        glob_ok = (c >= 3 * SQ) & (c < 3 * SQ + NG) & (my >= 2)
        mask = band_ok | glob_ok

        zpad = jnp.zeros((SQ - NG, DM), jnp.bfloat16)
        for b in range(B):
            k_cat = jnp.concatenate(
                [halobuf[0, b][:, :DM], k_ref[b].astype(jnp.bfloat16),
                 halobuf[1, b][:, :DM], gkv[0, b], zpad], axis=0)
            v_cat = jnp.concatenate(
                [halobuf[0, b][:, DM:], v_ref[b].astype(jnp.bfloat16),
                 halobuf[1, b][:, DM:], gkv[1, b], zpad], axis=0)
            ctx_heads = []
            for hh in range(HQ):
                q_h = q_all[b][:, hh * DH:(hh + 1) * DH].astype(jnp.bfloat16)
                k_h = k_cat[:, hh * DH:(hh + 1) * DH]
                v_h = v_cat[:, hh * DH:(hh + 1) * DH]
                s = lax.dot_general(
                    q_h, k_h, (((1,), (1,)), ((), ())),
                    preferred_element_type=jnp.float32) * SCALE
                s = jnp.where(mask, s, NEG)
                m = jnp.max(s, axis=1, keepdims=True)
                w = jnp.exp(s - m)
                w = w / jnp.sum(w, axis=1, keepdims=True)
                ctx_heads.append(jnp.dot(w.astype(jnp.bfloat16), v_h,
                                         preferred_element_type=jnp.float32))
            ctx = jnp.concatenate(ctx_heads, axis=-1)
            out_ref[b] = jnp.dot(ctx, wo_ref[...],
                                 preferred_element_type=jnp.float32)

        @pl.when(my == 0)
        def _():
            for j in range(3):
                pltpu.make_async_remote_copy(
                    src_ref=pown, dst_ref=pprt.at[j],
                    send_sem=psend.at[0], recv_sem=precv.at[j],
                    device_id=(my,), device_id_type=MESH).wait_recv()
            for b in range(B):
                plane0_acc, M0, L0 = _merge_packed(
                    [pown[b]] + [pplane[j, b] for j in range(3)])
                acc_h, M, L = _merge_packed(
                    [_pack(plane0_acc, M0, L0)]
                    + [pprt[j, b] for j in range(3)])
                ctx_g = jnp.concatenate(
                    [acc_h[hh] / L[:, hh:hh + 1] for hh in range(HQ)],
                    axis=-1)
                out_ref[b, 0:NG] = jnp.dot(ctx_g, wo_ref[...],
                                           preferred_element_type=jnp.float32)

        @pl.when(my < N_DEV - 1)
        def _():
            pltpu.make_async_remote_copy(
                src_ref=kvown, dst_ref=halobuf.at[0],
                send_sem=halo_send.at[0], recv_sem=halo_recv.at[0],
                device_id=(my,), device_id_type=MESH).wait_send()

        @pl.when(my > 0)
        def _():
            pltpu.make_async_remote_copy(
                src_ref=kvown, dst_ref=halobuf.at[1],
                send_sem=halo_send.at[1], recv_sem=halo_recv.at[1],
                device_id=(my,), device_id_type=MESH).wait_send()

        @pl.when(in4 != 0)
        def _():
            pltpu.make_async_remote_copy(
                src_ref=pown, dst_ref=pplane.at[0],
                send_sem=plsend.at[0], recv_sem=plrecv.at[0],
                device_id=(my,), device_id_type=MESH).wait_send()

        @pl.when((in4 == 0) & (my > 0))
        def _():
            pltpu.make_async_remote_copy(
                src_ref=pown, dst_ref=pprt.at[0],
                send_sem=psend.at[0], recv_sem=precv.at[0],
                device_id=(my,), device_id_type=MESH).wait_send()

        @pl.when(my == 0)
        def _():
            for q in range(1, N_DEV):
                pltpu.make_async_remote_copy(
                    src_ref=gq, dst_ref=gq,
                    send_sem=gqsend.at[q - 1], recv_sem=gqrecv.at[0],
                    device_id=(my,), device_id_type=MESH).wait_send()
                pltpu.make_async_remote_copy(
                    src_ref=gkv, dst_ref=gkv,
                    send_sem=gkvsend.at[q - 1], recv_sem=gkvrecv.at[0],
                    device_id=(my,), device_id_type=MESH).wait_send()

    return pl.pallas_call(
        body,
        out_shape=jax.ShapeDtypeStruct((B, SQ, DX), jnp.float32),
        in_specs=[pl.BlockSpec(memory_space=pltpu.VMEM)] * 5,
        out_specs=pl.BlockSpec(memory_space=pltpu.VMEM),
        scratch_shapes=[
            pltpu.VMEM((B, SQ, 2 * DM), jnp.bfloat16),
            pltpu.VMEM((2, B, SQ, 2 * DM), jnp.bfloat16),
            pltpu.VMEM((B, NG, DM), jnp.bfloat16),
            pltpu.VMEM((2, B, NG, DM), jnp.bfloat16),
            pltpu.VMEM((B, NG, PW), jnp.float32),
            pltpu.VMEM((3, B, NG, PW), jnp.float32),
            pltpu.VMEM((3, B, NG, PW), jnp.float32),
            pltpu.SemaphoreType.DMA((2,)),
            pltpu.SemaphoreType.DMA((2,)),
            pltpu.SemaphoreType.DMA((N_DEV - 1,)),
            pltpu.SemaphoreType.DMA((1,)),
            pltpu.SemaphoreType.DMA((N_DEV - 1,)),
            pltpu.SemaphoreType.DMA((1,)),
            pltpu.SemaphoreType.DMA((1,)),
            pltpu.SemaphoreType.DMA((3,)),
            pltpu.SemaphoreType.DMA((1,)),
            pltpu.SemaphoreType.DMA((3,)),
        ],
        compiler_params=pltpu.CompilerParams(collective_id=0),
    )(x, Wq, k2, v2, Wo)


# device time: 25964 ns/iter; 1.8897x vs baseline; 1.1232x over previous
import jax
import jax.numpy as jnp
from jax import lax
from jax.experimental import pallas as pl
from jax.experimental.pallas import tpu as pltpu

N_DEV = 16
B = 2
SQ = 128
HQ = 4
DH = 64
DM = HQ * DH
DX = 512
SKV = N_DEV * SQ
NG = 32
SCALE = 0.125
NEG = -1e9
PW = 384


def kernel(x, Wq, K_ext, V_ext, Wo):
    k2 = K_ext.reshape(B, SQ, DM)
    v2 = V_ext.reshape(B, SQ, DM)

    def body(x_ref, wq_ref, k_ref, v_ref, wo_ref, out_ref,
             kvown, halobuf, gq, gkv, pown, pprt,
             halo_send, halo_recv, gqsend, gqrecv, gkvsend, gkvrecv,
             psend, precv):
        my = lax.axis_index("i")
        MESH = pl.DeviceIdType.MESH

        bsem = pltpu.get_barrier_semaphore()

        @pl.when(my == 0)
        def _():
            for q in range(1, N_DEV):
                pl.semaphore_signal(bsem, inc=1, device_id=(q,),
                                    device_id_type=MESH)

        @pl.when(my != 0)
        def _():
            pl.semaphore_signal(bsem, inc=1, device_id=(0,),
                                device_id_type=MESH)

        @pl.when(my > 1)
        def _():
            pl.semaphore_signal(bsem, inc=1, device_id=(my - 1,),
                                device_id_type=MESH)

        @pl.when((my > 0) & (my < N_DEV - 1))
        def _():
            pl.semaphore_signal(bsem, inc=1, device_id=(my + 1,),
                                device_id_type=MESH)

        @pl.when(my == 0)
        def _():
            pl.semaphore_wait(bsem, N_DEV - 1)

        @pl.when((my == 1) | (my == N_DEV - 1))
        def _():
            pl.semaphore_wait(bsem, 2)

        @pl.when((my > 1) & (my < N_DEV - 1))
        def _():
            pl.semaphore_wait(bsem, 3)

        @pl.when(my == 0)
        def _():
            qg_rows = [jnp.dot(x_ref[b][0:NG, :], wq_ref[...],
                               preferred_element_type=jnp.float32)
                       for b in range(B)]
            for b in range(B):
                gq[b] = qg_rows[b].astype(jnp.bfloat16)
            for q in range(1, N_DEV):
                pltpu.make_async_remote_copy(
                    src_ref=gq, dst_ref=gq,
                    send_sem=gqsend.at[q - 1], recv_sem=gqrecv.at[0],
                    device_id=(q,), device_id_type=MESH).start()

        kvown[:, :, 0:DM] = k_ref[...].astype(jnp.bfloat16)
        kvown[:, :, DM:] = v_ref[...].astype(jnp.bfloat16)

        @pl.when(my == 0)
        def _():
            halobuf[0] = jnp.zeros((B, SQ, 2 * DM), jnp.bfloat16)

        @pl.when(my == N_DEV - 1)
        def _():
            halobuf[1] = jnp.zeros((B, SQ, 2 * DM), jnp.bfloat16)

        @pl.when(my < N_DEV - 1)
        def _():
            pltpu.make_async_remote_copy(
                src_ref=kvown, dst_ref=halobuf.at[0],
                send_sem=halo_send.at[0], recv_sem=halo_recv.at[0],
                device_id=(my + 1,), device_id_type=MESH).start()

        @pl.when(my > 0)
        def _():
            pltpu.make_async_remote_copy(
                src_ref=kvown, dst_ref=halobuf.at[1],
                send_sem=halo_send.at[1], recv_sem=halo_recv.at[1],
                device_id=(my - 1,), device_id_type=MESH).start()

        q_all = [jnp.dot(x_ref[b], wq_ref[...],
                         preferred_element_type=jnp.float32)
                 for b in range(B)]

        @pl.when(my == 0)
        def _():
            for b in range(B):
                gkv[0, b] = k_ref[b, 0:NG, :].astype(jnp.bfloat16)
                gkv[1, b] = v_ref[b, 0:NG, :].astype(jnp.bfloat16)
            for q in range(1, N_DEV):
                pltpu.make_async_remote_copy(
                    src_ref=gkv, dst_ref=gkv,
                    send_sem=gkvsend.at[q - 1], recv_sem=gkvrecv.at[0],
                    device_id=(q,), device_id_type=MESH).start()

        @pl.when(my != 0)
        def _():
            pltpu.make_async_remote_copy(
                src_ref=gq, dst_ref=gq,
                send_sem=gqsend.at[0], recv_sem=gqrecv.at[0],
                device_id=(0,), device_id_type=MESH).wait_recv()

        for b in range(B):
            accs, ms, ls = [], [], []
            for hh in range(HQ):
                qg_h = gq[b][:, hh * DH:(hh + 1) * DH]
                k_h = k_ref[b][:, hh * DH:(hh + 1) * DH].astype(jnp.bfloat16)
                v_h = v_ref[b][:, hh * DH:(hh + 1) * DH].astype(jnp.bfloat16)
                s = lax.dot_general(
                    qg_h, k_h, (((1,), (1,)), ((), ())),
                    preferred_element_type=jnp.float32) * SCALE
                m = jnp.max(s, axis=1, keepdims=True)
                m = m.astype(jnp.bfloat16).astype(jnp.float32)
                w = jnp.exp(s - m)
                ls.append(jnp.sum(w, axis=1, keepdims=True))
                ms.append(m)
                accs.append(jnp.dot(w.astype(jnp.bfloat16), v_h,
                                    preferred_element_type=jnp.float32))
            pown[b] = jnp.concatenate(
                accs + ms + ls + [jnp.zeros((NG, PW - DM - 2 * HQ),
                                            jnp.float32)],
                axis=1).astype(jnp.bfloat16)

        @pl.when(my != 0)
        def _():
            pltpu.make_async_remote_copy(
                src_ref=pown, dst_ref=pprt.at[my - 1],
                send_sem=psend.at[0], recv_sem=precv.at[my - 1],
                device_id=(0,), device_id_type=MESH).start()

        @pl.when(my > 0)
        def _():
            pltpu.make_async_remote_copy(
                src_ref=kvown, dst_ref=halobuf.at[0],
                send_sem=halo_send.at[0], recv_sem=halo_recv.at[0],
                device_id=(my,), device_id_type=MESH).wait_recv()

        @pl.when(my < N_DEV - 1)
        def _():
            pltpu.make_async_remote_copy(
                src_ref=kvown, dst_ref=halobuf.at[1],
                send_sem=halo_send.at[1], recv_sem=halo_recv.at[1],
                device_id=(my,), device_id_type=MESH).wait_recv()

        @pl.when(my != 0)
        def _():
            pltpu.make_async_remote_copy(
                src_ref=gkv, dst_ref=gkv,
                send_sem=gkvsend.at[0], recv_sem=gkvrecv.at[0],
                device_id=(0,), device_id_type=MESH).wait_recv()

        r = lax.broadcasted_iota(jnp.int32, (SQ, 4 * SQ), 0)
        c = lax.broadcasted_iota(jnp.int32, (SQ, 4 * SQ), 1)
        qi = my * SQ + r
        ki = (my - 1) * SQ + c
        band_ok = ((c < 3 * SQ) & (ki >= 0) & (ki < SKV)
                   & ((jnp.abs(qi - ki) <= 128) | (ki < NG) | (qi < NG)))
        glob_ok = (c >= 3 * SQ) & (c < 3 * SQ + NG) & (my >= 2)
        mask = band_ok | glob_ok

        zpad = jnp.zeros((SQ - NG, DM), jnp.bfloat16)
        for b in range(B):
            k_cat = jnp.concatenate(
                [halobuf[0, b][:, :DM], k_ref[b].astype(jnp.bfloat16),
                 halobuf[1, b][:, :DM], gkv[0, b], zpad], axis=0)
            v_cat = jnp.concatenate(
                [halobuf[0, b][:, DM:], v_ref[b].astype(jnp.bfloat16),
                 halobuf[1, b][:, DM:], gkv[1, b], zpad], axis=0)
            ctx_heads = []
            for hh in range(HQ):
                q_h = q_all[b][:, hh * DH:(hh + 1) * DH].astype(jnp.bfloat16)
                k_h = k_cat[:, hh * DH:(hh + 1) * DH]
                v_h = v_cat[:, hh * DH:(hh + 1) * DH]
                s = lax.dot_general(
                    q_h, k_h, (((1,), (1,)), ((), ())),
                    preferred_element_type=jnp.float32) * SCALE
                s = jnp.where(mask, s, NEG)
                m = jnp.max(s, axis=1, keepdims=True)
                w = jnp.exp(s - m)
                w = w / jnp.sum(w, axis=1, keepdims=True)
                ctx_heads.append(jnp.dot(w.astype(jnp.bfloat16), v_h,
                                         preferred_element_type=jnp.float32))
            ctx = jnp.concatenate(ctx_heads, axis=-1)
            out_ref[b] = jnp.dot(ctx, wo_ref[...],
                                 preferred_element_type=jnp.float32)

        @pl.when(my == 0)
        def _():
            for j in range(N_DEV - 1):
                pltpu.make_async_remote_copy(
                    src_ref=pown, dst_ref=pprt.at[j],
                    send_sem=psend.at[0], recv_sem=precv.at[j],
                    device_id=(my,), device_id_type=MESH).wait_recv()
            for b in range(B):
                parts = ([pown[b].astype(jnp.float32)]
                         + [pprt[j, b].astype(jnp.float32)
                            for j in range(N_DEV - 1)])
                m_all = [p_[:, DM:DM + HQ] for p_ in parts]
                l_all = [p_[:, DM + HQ:DM + 2 * HQ] for p_ in parts]
                M = m_all[0]
                for mm in m_all[1:]:
                    M = jnp.maximum(M, mm)
                coefs = [jnp.exp(mm - M) for mm in m_all]
                L = coefs[0] * l_all[0]
                for cf, ll in zip(coefs[1:], l_all[1:]):
                    L = L + cf * ll
                ctx_g_heads = []
                for hh in range(HQ):
                    num = (parts[0][:, hh * DH:(hh + 1) * DH]
                           * coefs[0][:, hh:hh + 1])
                    for p_, cf in zip(parts[1:], coefs[1:]):
                        num = num + (p_[:, hh * DH:(hh + 1) * DH]
                                     * cf[:, hh:hh + 1])
                    ctx_g_heads.append(num / L[:, hh:hh + 1])
                ctx_g = jnp.concatenate(ctx_g_heads, axis=-1)
                out_ref[b, 0:NG] = jnp.dot(ctx_g, wo_ref[...],
                                           preferred_element_type=jnp.float32)

        @pl.when(my < N_DEV - 1)
        def _():
            pltpu.make_async_remote_copy(
                src_ref=kvown, dst_ref=halobuf.at[0],
                send_sem=halo_send.at[0], recv_sem=halo_recv.at[0],
                device_id=(my,), device_id_type=MESH).wait_send()

        @pl.when(my > 0)
        def _():
            pltpu.make_async_remote_copy(
                src_ref=kvown, dst_ref=halobuf.at[1],
                send_sem=halo_send.at[1], recv_sem=halo_recv.at[1],
                device_id=(my,), device_id_type=MESH).wait_send()
            pltpu.make_async_remote_copy(
                src_ref=pown, dst_ref=pprt.at[0],
                send_sem=psend.at[0], recv_sem=precv.at[0],
                device_id=(my,), device_id_type=MESH).wait_send()

        @pl.when(my == 0)
        def _():
            for q in range(1, N_DEV):
                pltpu.make_async_remote_copy(
                    src_ref=gq, dst_ref=gq,
                    send_sem=gqsend.at[q - 1], recv_sem=gqrecv.at[0],
                    device_id=(my,), device_id_type=MESH).wait_send()
                pltpu.make_async_remote_copy(
                    src_ref=gkv, dst_ref=gkv,
                    send_sem=gkvsend.at[q - 1], recv_sem=gkvrecv.at[0],
                    device_id=(my,), device_id_type=MESH).wait_send()

    return pl.pallas_call(
        body,
        out_shape=jax.ShapeDtypeStruct((B, SQ, DX), jnp.float32),
        in_specs=[pl.BlockSpec(memory_space=pltpu.VMEM)] * 5,
        out_specs=pl.BlockSpec(memory_space=pltpu.VMEM),
        scratch_shapes=[
            pltpu.VMEM((B, SQ, 2 * DM), jnp.bfloat16),
            pltpu.VMEM((2, B, SQ, 2 * DM), jnp.bfloat16),
            pltpu.VMEM((B, NG, DM), jnp.bfloat16),
            pltpu.VMEM((2, B, NG, DM), jnp.bfloat16),
            pltpu.VMEM((B, NG, PW), jnp.bfloat16),
            pltpu.VMEM((N_DEV - 1, B, NG, PW), jnp.bfloat16),
            pltpu.SemaphoreType.DMA((2,)),
            pltpu.SemaphoreType.DMA((2,)),
            pltpu.SemaphoreType.DMA((N_DEV - 1,)),
            pltpu.SemaphoreType.DMA((1,)),
            pltpu.SemaphoreType.DMA((N_DEV - 1,)),
            pltpu.SemaphoreType.DMA((1,)),
            pltpu.SemaphoreType.DMA((1,)),
            pltpu.SemaphoreType.DMA((N_DEV - 1,)),
        ],
        compiler_params=pltpu.CompilerParams(collective_id=0),
    )(x, Wq, k2, v2, Wo)


# device time: 24344 ns/iter; 2.0154x vs baseline; 1.0665x over previous
import jax
import jax.numpy as jnp
from jax import lax
from jax.experimental import pallas as pl
from jax.experimental.pallas import tpu as pltpu

N_DEV = 16
B = 2
SQ = 128
HQ = 4
DH = 64
DM = HQ * DH
DX = 512
SKV = N_DEV * SQ
NG = 32
SCALE = 0.125
NEG = -1e9
PW = 384


def kernel(x, Wq, K_ext, V_ext, Wo):
    k2 = K_ext.reshape(B, SQ, DM)
    v2 = V_ext.reshape(B, SQ, DM)

    def body(x_ref, wq_ref, k_ref, v_ref, wo_ref, out_ref,
             kvown, halobuf, gq, gkv, pown, pprt,
             halo_send, halo_recv, gqsend, gqrecv, gkvsend, gkvrecv,
             psend, precv):
        my = lax.axis_index("i")
        MESH = pl.DeviceIdType.MESH

        bsem = pltpu.get_barrier_semaphore()

        @pl.when(my == 0)
        def _():
            for q in range(1, N_DEV):
                pl.semaphore_signal(bsem, inc=1, device_id=(q,),
                                    device_id_type=MESH)

        @pl.when(my != 0)
        def _():
            pl.semaphore_signal(bsem, inc=1, device_id=(0,),
                                device_id_type=MESH)

        @pl.when(my > 1)
        def _():
            pl.semaphore_signal(bsem, inc=1, device_id=(my - 1,),
                                device_id_type=MESH)

        @pl.when((my > 0) & (my < N_DEV - 1))
        def _():
            pl.semaphore_signal(bsem, inc=1, device_id=(my + 1,),
                                device_id_type=MESH)

        kvown[:, :, 0:DM] = k_ref[...].astype(jnp.bfloat16)
        kvown[:, :, DM:] = v_ref[...].astype(jnp.bfloat16)

        @pl.when(my == 0)
        def _():
            halobuf[0] = jnp.zeros((B, SQ, 2 * DM), jnp.bfloat16)

        @pl.when(my == N_DEV - 1)
        def _():
            halobuf[1] = jnp.zeros((B, SQ, 2 * DM), jnp.bfloat16)

        q_all = [jnp.dot(x_ref[b], wq_ref[...],
                         preferred_element_type=jnp.float32)
                 for b in range(B)]

        @pl.when(my == 0)
        def _():
            for b in range(B):
                gq[b] = q_all[b][0:NG, :].astype(jnp.bfloat16)
                gkv[0, b] = k_ref[b, 0:NG, :].astype(jnp.bfloat16)
                gkv[1, b] = v_ref[b, 0:NG, :].astype(jnp.bfloat16)

        @pl.when(my == 0)
        def _():
            pl.semaphore_wait(bsem, N_DEV - 1)

        @pl.when((my == 1) | (my == N_DEV - 1))
        def _():
            pl.semaphore_wait(bsem, 2)

        @pl.when((my > 1) & (my < N_DEV - 1))
        def _():
            pl.semaphore_wait(bsem, 3)

        @pl.when(my == 0)
        def _():
            for q in range(N_DEV - 1, 0, -1):
                pltpu.make_async_remote_copy(
                    src_ref=gq, dst_ref=gq,
                    send_sem=gqsend.at[q - 1], recv_sem=gqrecv.at[0],
                    device_id=(q,), device_id_type=MESH).start()

        @pl.when(my < N_DEV - 1)
        def _():
            pltpu.make_async_remote_copy(
                src_ref=kvown, dst_ref=halobuf.at[0],
                send_sem=halo_send.at[0], recv_sem=halo_recv.at[0],
                device_id=(my + 1,), device_id_type=MESH).start()

        @pl.when(my > 0)
        def _():
            pltpu.make_async_remote_copy(
                src_ref=kvown, dst_ref=halobuf.at[1],
                send_sem=halo_send.at[1], recv_sem=halo_recv.at[1],
                device_id=(my - 1,), device_id_type=MESH).start()

        @pl.when(my == 0)
        def _():
            for q in range(N_DEV - 1, 0, -1):
                pltpu.make_async_remote_copy(
                    src_ref=gkv, dst_ref=gkv,
                    send_sem=gkvsend.at[q - 1], recv_sem=gkvrecv.at[0],
                    device_id=(q,), device_id_type=MESH).start()

        @pl.when(my != 0)
        def _():
            pltpu.make_async_remote_copy(
                src_ref=gq, dst_ref=gq,
                send_sem=gqsend.at[0], recv_sem=gqrecv.at[0],
                device_id=(0,), device_id_type=MESH).wait_recv()

        for b in range(B):
            accs, ms, ls = [], [], []
            for hh in range(HQ):
                qg_h = gq[b][:, hh * DH:(hh + 1) * DH]
                k_h = k_ref[b][:, hh * DH:(hh + 1) * DH].astype(jnp.bfloat16)
                v_h = v_ref[b][:, hh * DH:(hh + 1) * DH].astype(jnp.bfloat16)
                s = lax.dot_general(
                    qg_h, k_h, (((1,), (1,)), ((), ())),
                    preferred_element_type=jnp.float32) * SCALE
                m = jnp.max(s, axis=1, keepdims=True)
                m = m.astype(jnp.bfloat16).astype(jnp.float32)
                w = jnp.exp(s - m)
                ls.append(jnp.sum(w, axis=1, keepdims=True))
                ms.append(m)
                accs.append(jnp.dot(w.astype(jnp.bfloat16), v_h,
                                    preferred_element_type=jnp.float32))
            pown[b] = jnp.concatenate(
                accs + ms + ls + [jnp.zeros((NG, PW - DM - 2 * HQ),
                                            jnp.float32)],
                axis=1).astype(jnp.bfloat16)

        @pl.when(my != 0)
        def _():
            pltpu.make_async_remote_copy(
                src_ref=pown, dst_ref=pprt.at[my - 1],
                send_sem=psend.at[0], recv_sem=precv.at[my - 1],
                device_id=(0,), device_id_type=MESH).start()

        @pl.when(my > 0)
        def _():
            pltpu.make_async_remote_copy(
                src_ref=kvown, dst_ref=halobuf.at[0],
                send_sem=halo_send.at[0], recv_sem=halo_recv.at[0],
                device_id=(my,), device_id_type=MESH).wait_recv()

        @pl.when(my < N_DEV - 1)
        def _():
            pltpu.make_async_remote_copy(
                src_ref=kvown, dst_ref=halobuf.at[1],
                send_sem=halo_send.at[1], recv_sem=halo_recv.at[1],
                device_id=(my,), device_id_type=MESH).wait_recv()

        @pl.when(my != 0)
        def _():
            pltpu.make_async_remote_copy(
                src_ref=gkv, dst_ref=gkv,
                send_sem=gkvsend.at[0], recv_sem=gkvrecv.at[0],
                device_id=(0,), device_id_type=MESH).wait_recv()

        r = lax.broadcasted_iota(jnp.int32, (SQ, 4 * SQ), 0)
        c = lax.broadcasted_iota(jnp.int32, (SQ, 4 * SQ), 1)
        qi = my * SQ + r
        ki = (my - 1) * SQ + c
        band_ok = ((c < 3 * SQ) & (ki >= 0) & (ki < SKV)
                   & ((jnp.abs(qi - ki) <= 128) | (ki < NG) | (qi < NG)))
        glob_ok = (c >= 3 * SQ) & (c < 3 * SQ + NG) & (my >= 2)
        mask = band_ok | glob_ok

        zpad = jnp.zeros((SQ - NG, DM), jnp.bfloat16)
        for b in range(B):
            k_cat = jnp.concatenate(
                [halobuf[0, b][:, :DM], k_ref[b].astype(jnp.bfloat16),
                 halobuf[1, b][:, :DM], gkv[0, b], zpad], axis=0)
            v_cat = jnp.concatenate(
                [halobuf[0, b][:, DM:], v_ref[b].astype(jnp.bfloat16),
                 halobuf[1, b][:, DM:], gkv[1, b], zpad], axis=0)
            ctx_heads = []
            for hh in range(HQ):
                q_h = q_all[b][:, hh * DH:(hh + 1) * DH].astype(jnp.bfloat16)
                k_h = k_cat[:, hh * DH:(hh + 1) * DH]
                v_h = v_cat[:, hh * DH:(hh + 1) * DH]
                s = lax.dot_general(
                    q_h, k_h, (((1,), (1,)), ((), ())),
                    preferred_element_type=jnp.float32) * SCALE
                s = jnp.where(mask, s, NEG)
                m = jnp.max(s, axis=1, keepdims=True)
                w = jnp.exp(s - m)
                w = w / jnp.sum(w, axis=1, keepdims=True)
                ctx_heads.append(jnp.dot(w.astype(jnp.bfloat16), v_h,
                                         preferred_element_type=jnp.float32))
            ctx = jnp.concatenate(ctx_heads, axis=-1)
            out_ref[b] = jnp.dot(ctx, wo_ref[...],
                                 preferred_element_type=jnp.float32)

        @pl.when(my == 0)
        def _():
            for j in range(N_DEV - 1):
                pltpu.make_async_remote_copy(
                    src_ref=pown, dst_ref=pprt.at[j],
                    send_sem=psend.at[0], recv_sem=precv.at[j],
                    device_id=(my,), device_id_type=MESH).wait_recv()
            for b in range(B):
                parts = ([pown[b].astype(jnp.float32)]
                         + [pprt[j, b].astype(jnp.float32)
                            for j in range(N_DEV - 1)])
                m_all = [p_[:, DM:DM + HQ] for p_ in parts]
                l_all = [p_[:, DM + HQ:DM + 2 * HQ] for p_ in parts]
                M = m_all[0]
                for mm in m_all[1:]:
                    M = jnp.maximum(M, mm)
                coefs = [jnp.exp(mm - M) for mm in m_all]
                L = coefs[0] * l_all[0]
                for cf, ll in zip(coefs[1:], l_all[1:]):
                    L = L + cf * ll
                ctx_g_heads = []
                for hh in range(HQ):
                    num = (parts[0][:, hh * DH:(hh + 1) * DH]
                           * coefs[0][:, hh:hh + 1])
                    for p_, cf in zip(parts[1:], coefs[1:]):
                        num = num + (p_[:, hh * DH:(hh + 1) * DH]
                                     * cf[:, hh:hh + 1])
                    ctx_g_heads.append(num / L[:, hh:hh + 1])
                ctx_g = jnp.concatenate(ctx_g_heads, axis=-1)
                out_ref[b, 0:NG] = jnp.dot(ctx_g, wo_ref[...],
                                           preferred_element_type=jnp.float32)

        @pl.when(my < N_DEV - 1)
        def _():
            pltpu.make_async_remote_copy(
                src_ref=kvown, dst_ref=halobuf.at[0],
                send_sem=halo_send.at[0], recv_sem=halo_recv.at[0],
                device_id=(my,), device_id_type=MESH).wait_send()

        @pl.when(my > 0)
        def _():
            pltpu.make_async_remote_copy(
                src_ref=kvown, dst_ref=halobuf.at[1],
                send_sem=halo_send.at[1], recv_sem=halo_recv.at[1],
                device_id=(my,), device_id_type=MESH).wait_send()
            pltpu.make_async_remote_copy(
                src_ref=pown, dst_ref=pprt.at[0],
                send_sem=psend.at[0], recv_sem=precv.at[0],
                device_id=(my,), device_id_type=MESH).wait_send()

        @pl.when(my == 0)
        def _():
            for q in range(1, N_DEV):
                pltpu.make_async_remote_copy(
                    src_ref=gq, dst_ref=gq,
                    send_sem=gqsend.at[q - 1], recv_sem=gqrecv.at[0],
                    device_id=(my,), device_id_type=MESH).wait_send()
                pltpu.make_async_remote_copy(
                    src_ref=gkv, dst_ref=gkv,
                    send_sem=gkvsend.at[q - 1], recv_sem=gkvrecv.at[0],
                    device_id=(my,), device_id_type=MESH).wait_send()

    return pl.pallas_call(
        body,
        out_shape=jax.ShapeDtypeStruct((B, SQ, DX), jnp.float32),
        in_specs=[pl.BlockSpec(memory_space=pltpu.VMEM)] * 5,
        out_specs=pl.BlockSpec(memory_space=pltpu.VMEM),
        scratch_shapes=[
            pltpu.VMEM((B, SQ, 2 * DM), jnp.bfloat16),
            pltpu.VMEM((2, B, SQ, 2 * DM), jnp.bfloat16),
            pltpu.VMEM((B, NG, DM), jnp.bfloat16),
            pltpu.VMEM((2, B, NG, DM), jnp.bfloat16),
            pltpu.VMEM((B, NG, PW), jnp.bfloat16),
            pltpu.VMEM((N_DEV - 1, B, NG, PW), jnp.bfloat16),
            pltpu.SemaphoreType.DMA((2,)),
            pltpu.SemaphoreType.DMA((2,)),
            pltpu.SemaphoreType.DMA((N_DEV - 1,)),
            pltpu.SemaphoreType.DMA((1,)),
            pltpu.SemaphoreType.DMA((N_DEV - 1,)),
            pltpu.SemaphoreType.DMA((1,)),
            pltpu.SemaphoreType.DMA((1,)),
            pltpu.SemaphoreType.DMA((N_DEV - 1,)),
        ],
        compiler_params=pltpu.CompilerParams(collective_id=0),
    )(x, Wq, k2, v2, Wo)
